# Initial kernel scaffold; baseline (speedup 1.0000x reference)
import jax
import jax.numpy as jnp
from jax.experimental import pallas as pl


def kernel(x):
    raise NotImplementedError("write your pallas kernel here")


# speedup vs baseline: 9.2895x; 9.2895x over previous
import math

import jax
import jax.numpy as jnp
from jax.experimental import pallas as pl
from jax.experimental.pallas import tpu as pltpu

_PCT = 0.1
_ROWS_PER_BLOCK = 8
_INT_MIN = -(2 ** 31)


def _select_mask_body(k, x_ref, o_ref, s_ref):
    int_min = jnp.int32(_INT_MIN)
    x = x_ref[...]
    bits = pltpu.bitcast(x, jnp.int32)
    bits = jnp.where(x == 0.0, jnp.int32(0), bits)
    s = jnp.where(bits < 0, ~bits ^ int_min, bits)
    s_ref[...] = s

    def body(i, cur):
        bit = 31 - i
        cand = cur | (jnp.int32(1) << bit)
        keys = s_ref[...]
        cnt = jnp.sum((keys >= (cand ^ int_min)).astype(jnp.int32),
                      axis=1, keepdims=True)
        return jnp.where(cnt >= k, cand, cur)

    nrows = x.shape[0]
    cur = jax.lax.fori_loop(0, 32, body,
                            jnp.zeros((nrows, 1), jnp.int32))
    thr = cur ^ int_min
    o_ref[...] = jnp.where(s_ref[...] >= thr, x, 0.0)


def kernel(x):
    n_rows, n_feat = x.shape
    k = max(1, math.ceil(n_feat * _PCT))
    rb = _ROWS_PER_BLOCK
    grid = (n_rows // rb,)

    import functools
    body = functools.partial(_select_mask_body, k)
    return pl.pallas_call(
        body,
        grid=grid,
        in_specs=[pl.BlockSpec((rb, n_feat), lambda i: (i, 0))],
        out_specs=pl.BlockSpec((rb, n_feat), lambda i: (i, 0)),
        out_shape=jax.ShapeDtypeStruct(x.shape, x.dtype),
        scratch_shapes=[pltpu.VMEM((rb, n_feat), jnp.int32)],
    )(x)
